# baseline (speedup 1.0000x reference)
import functools

import jax
import jax.numpy as jnp
from jax import lax
from jax.experimental import pallas as pl
from jax.experimental.pallas import tpu as pltpu
from jax.experimental.pallas import tpu_sc as plsc

_THRESH = 0.7
_N_MIN = 131072
_N_CLASSES = 19
_NPIX = 8 * 512 * 512
_NW = 32
_CHUNK = _NPIX // _NW
_NB1 = 2048
_NB3 = 1024
_CPAD = 32
_TH07_BITS = 0x3F333333


def _phase_a(logits, labels, *, interpret=False):
    N, C, H, W = logits.shape
    BH = 64
    grid = (N, H // BH)

    def body(lg_ref, lb_ref, pick_ref, nll_ref, c07_ref, acc_ref):
        x = lg_ref[0]
        lab = lb_ref[0]
        m = jnp.max(x, axis=0)
        e = jnp.exp(x - m[None])
        s = jnp.sum(e, axis=0)
        cls = lax.broadcasted_iota(jnp.int32, (C, BH, W), 0)
        onehot = cls == lab[None]
        picked = jnp.sum(jnp.where(onehot, x, 0.0), axis=0)
        nll = (m - picked) + jnp.log(s)
        pick = jnp.exp(-nll)
        pick_ref[0] = lax.bitcast_convert_type(pick, jnp.int32)
        nll_ref[0] = nll
        nle = jnp.sum((pick <= jnp.float32(_THRESH)).astype(jnp.float32))
        valid07 = nll >= -jnp.log(jnp.float32(_THRESH))
        contrib = jnp.sum(
            jnp.where(onehot & valid07[None], 1.0, 0.0), axis=1)
        contrib32 = jnp.concatenate(
            [contrib, jnp.zeros((_CPAD - C, W), jnp.float32)], axis=0)

        @pl.when((pl.program_id(0) == 0) & (pl.program_id(1) == 0))
        def _():
            c07_ref[0, 0] = 0.0
            acc_ref[...] = jnp.zeros((_CPAD, W), jnp.float32)
        c07_ref[0, 0] += nle
        acc_ref[...] += contrib32

    return pl.pallas_call(
        body,
        grid=grid,
        in_specs=[
            pl.BlockSpec((1, C, BH, W), lambda n, h: (n, 0, h, 0)),
            pl.BlockSpec((1, BH, W), lambda n, h: (n, h, 0)),
        ],
        out_specs=[
            pl.BlockSpec((1, BH, W), lambda n, h: (n, h, 0)),
            pl.BlockSpec((1, BH, W), lambda n, h: (n, h, 0)),
            pl.BlockSpec(memory_space=pltpu.MemorySpace.SMEM,
                         index_map=lambda n, h: (0, 0)),
            pl.BlockSpec((_CPAD, W), lambda n, h: (0, 0)),
        ],
        out_shape=[
            jax.ShapeDtypeStruct((N, H, W), jnp.int32),
            jax.ShapeDtypeStruct((N, H, W), jnp.float32),
            jax.ShapeDtypeStruct((1, 1), jnp.float32),
            jax.ShapeDtypeStruct((_CPAD, W), jnp.float32),
        ],
        interpret=interpret,
    )(logits, labels)


def _hist_pass(shift, nbins, prev_shift, *, interpret=False):
    mesh = plsc.VectorSubcoreMesh(core_axis_name="c", subcore_axis_name="s", num_cores=2, num_subcores=16)
    masked = prev_shift is not None

    scratch = [
        pltpu.VMEM((_CHUNK,), jnp.int32),
        pltpu.VMEM((16 * nbins,), jnp.int32),
        pltpu.VMEM((nbins,), jnp.int32),
    ]
    if masked:
        scratch.append(pltpu.VMEM((128,), jnp.int32))

    def body(picks_hbm, *rest):
        if masked:
            state_hbm, out_hbm, chunk_v, hist_v, row_v, pref_v = rest
        else:
            out_hbm, chunk_v, hist_v, row_v = rest
        wid = lax.axis_index("s") * 2 + lax.axis_index("c")
        base = wid * _CHUNK
        pltpu.sync_copy(picks_hbm.at[pl.ds(base, _CHUNK)], chunk_v)
        if masked:
            pltpu.sync_copy(state_hbm.at[0], pref_v)
            prefv = pref_v[pl.ds(0, 16)]

        zeros16 = jnp.zeros((16,), jnp.int32)

        def zbody(j, _):
            hist_v[pl.ds(j * 16, 16)] = zeros16
            return 0
        lax.fori_loop(0, (16 * nbins) // 16, zbody, 0, unroll=8)

        iota = lax.iota(jnp.int32, 16)
        lane_off = iota * nbins
        ones16 = jnp.ones((16,), jnp.int32)
        shift_v = jnp.full((16,), shift, jnp.int32)
        maskbins = jnp.full((16,), nbins - 1, jnp.int32)
        if masked:
            pshift_v = jnp.full((16,), prev_shift, jnp.int32)

        def hbody(i, _):
            bits = chunk_v[pl.ds(i * 16, 16)]
            b = (lax.shift_right_logical(bits, shift_v) & maskbins) + lane_off
            if masked:
                m = lax.shift_right_logical(bits, pshift_v) == prefv
                plsc.addupdate_scatter(hist_v, [b], ones16, mask=m)
            else:
                plsc.addupdate_scatter(hist_v, [b], ones16)
            return 0
        lax.fori_loop(0, _CHUNK // 16, hbody, 0, unroll=8)

        def sbody(j, _):
            acc = hist_v[pl.ds(j * 16, 16)]
            for l in range(1, 16):
                acc = acc + hist_v[pl.ds(l * nbins + j * 16, 16)]
            row_v[pl.ds(j * 16, 16)] = acc
            return 0
        lax.fori_loop(0, nbins // 16, sbody, 0, unroll=4)

        pltpu.sync_copy(row_v, out_hbm.at[wid])

    return functools.partial(
        pl.kernel,
        out_type=jax.ShapeDtypeStruct((_NW, nbins), jnp.int32),
        mesh=mesh,
        scratch_types=scratch,
        compiler_params=pltpu.CompilerParams(needs_layout_passes=False),
        interpret=interpret,
    )(body)


def _find_bin(width, *, final=False, interpret=False):
    def body(hist_ref, st_ref, out_ref):
        hf = hist_ref[...].astype(jnp.float32)
        h = jnp.sum(hf, axis=0)
        R = h.shape[0]
        r_i = st_ref[1, 0]
        p_i = st_ref[0, 0]
        r_f = r_i.astype(jnp.float32)
        tri = (lax.broadcasted_iota(jnp.int32, (128, 128), 0)
               <= lax.broadcasted_iota(jnp.int32, (128, 128), 1))
        rowcum = jax.lax.dot(h, tri.astype(jnp.float32),
                             preferred_element_type=jnp.float32)
        rowtot = rowcum[:, 127:128]
        stri = (lax.broadcasted_iota(jnp.int32, (R, R), 1)
                < lax.broadcasted_iota(jnp.int32, (R, R), 0))
        prev = jax.lax.dot(stri.astype(jnp.float32), rowtot,
                           preferred_element_type=jnp.float32)
        c = rowcum + prev
        mask = c <= r_f
        bin_i = jnp.sum(mask.astype(jnp.int32))
        cbefore = jnp.max(jnp.where(mask, c, 0.0))
        newrank = r_i - cbefore.astype(jnp.int32)
        newprefix = lax.shift_left(p_i, width) | bin_i
        if final:
            q = lax.bitcast_convert_type(newprefix, jnp.float32)
            th = jnp.maximum(q, jnp.float32(_THRESH))
            thr = -jnp.log(th)
            for j in range(16):
                out_ref[0, j] = thr
        else:
            for j in range(16):
                out_ref[0, j] = newprefix
                out_ref[1, j] = newrank

    out_dtype = jnp.float32 if final else jnp.int32
    return pl.pallas_call(
        body,
        in_specs=[
            pl.BlockSpec(memory_space=pltpu.MemorySpace.VMEM),
            pl.BlockSpec(memory_space=pltpu.MemorySpace.SMEM),
        ],
        out_specs=pl.BlockSpec(memory_space=pltpu.MemorySpace.SMEM),
        out_shape=jax.ShapeDtypeStruct((8, 128), out_dtype),
        interpret=interpret,
    )


def _class_sums(*, interpret=False):
    mesh = plsc.VectorSubcoreMesh(core_axis_name="c", subcore_axis_name="s",
                                  num_cores=2, num_subcores=16)
    ROWS = 32

    def body(labels_hbm, nll_hbm, thr_hbm,
             cnt_out, sum_out,
             lab_v, nll_v, thr_v, ca_v, cb_v, sa_v, sb_v):
        wid = lax.axis_index("s") * 2 + lax.axis_index("c")
        rbase = wid * 128
        pltpu.sync_copy(thr_hbm.at[0], thr_v)
        thrv = thr_v[pl.ds(0, 16)]

        zeros16 = jnp.zeros((16,), jnp.float32)
        for i in range(_CPAD):
            ca_v[pl.ds(i * 16, 16)] = zeros16
            cb_v[pl.ds(i * 16, 16)] = zeros16
            sa_v[pl.ds(i * 16, 16)] = zeros16
            sb_v[pl.ds(i * 16, 16)] = zeros16

        iota = lax.iota(jnp.int32, 16)
        ones16 = jnp.ones((16,), jnp.float32)

        def outer(k, _):
            pltpu.sync_copy(labels_hbm.at[pl.ds(rbase + k * ROWS, ROWS)], lab_v)
            pltpu.sync_copy(nll_hbm.at[pl.ds(rbase + k * ROWS, ROWS)], nll_v)

            def row(r, _):
                for j in range(32):
                    lb = lab_v[r, pl.ds(j * 16, 16)]
                    nl = nll_v[r, pl.ds(j * 16, 16)]
                    valid = nl >= thrv
                    idx = lb * 16 + iota
                    if j % 2 == 0:
                        plsc.addupdate_scatter(ca_v, [idx], ones16,
                                               mask=valid)
                        plsc.addupdate_scatter(sa_v, [idx], nl, mask=valid)
                    else:
                        plsc.addupdate_scatter(cb_v, [idx], ones16,
                                               mask=valid)
                        plsc.addupdate_scatter(sb_v, [idx], nl, mask=valid)
                return 0
            lax.fori_loop(0, ROWS, row, 0)
            return 0
        lax.fori_loop(0, 128 // ROWS, outer, 0)

        def merge(i, _):
            ca_v[pl.ds(i * 16, 16)] += cb_v[pl.ds(i * 16, 16)]
            sa_v[pl.ds(i * 16, 16)] += sb_v[pl.ds(i * 16, 16)]
            return 0
        lax.fori_loop(0, _CPAD, merge, 0, unroll=8)

        pltpu.sync_copy(ca_v, cnt_out.at[wid])
        pltpu.sync_copy(sa_v, sum_out.at[wid])

    return functools.partial(
        pl.kernel,
        out_type=[
            jax.ShapeDtypeStruct((_NW, _CPAD * 16), jnp.float32),
            jax.ShapeDtypeStruct((_NW, _CPAD * 16), jnp.float32),
        ],
        mesh=mesh,
        scratch_types=[
            pltpu.VMEM((ROWS, 512), jnp.int32),
            pltpu.VMEM((ROWS, 512), jnp.float32),
            pltpu.VMEM((128,), jnp.float32),
            pltpu.VMEM((_CPAD * 16,), jnp.float32),
            pltpu.VMEM((_CPAD * 16,), jnp.float32),
            pltpu.VMEM((_CPAD * 16,), jnp.float32),
            pltpu.VMEM((_CPAD * 16,), jnp.float32),
        ],
        compiler_params=pltpu.CompilerParams(needs_layout_passes=False),
        interpret=interpret,
    )(body)


def _class_sums_only(*, interpret=False):
    mesh = plsc.VectorSubcoreMesh(core_axis_name="c", subcore_axis_name="s",
                                  num_cores=2, num_subcores=16)
    ROWS = 32

    def body(labels_hbm, nll_hbm, thr_hbm, sum_out,
             lab_v, nll_v, thr_v, sa_v, sb_v):
        wid = lax.axis_index("s") * 2 + lax.axis_index("c")
        rbase = wid * 128
        pltpu.sync_copy(thr_hbm.at[0], thr_v)
        thrv = thr_v[pl.ds(0, 16)]

        zeros16 = jnp.zeros((16,), jnp.float32)
        for i in range(_CPAD):
            sa_v[pl.ds(i * 16, 16)] = zeros16
            sb_v[pl.ds(i * 16, 16)] = zeros16

        iota = lax.iota(jnp.int32, 16)

        def outer(k, _):
            pltpu.sync_copy(labels_hbm.at[pl.ds(rbase + k * ROWS, ROWS)],
                            lab_v)
            pltpu.sync_copy(nll_hbm.at[pl.ds(rbase + k * ROWS, ROWS)], nll_v)

            def row(r, _):
                for j in range(32):
                    lb = lab_v[r, pl.ds(j * 16, 16)]
                    nl = nll_v[r, pl.ds(j * 16, 16)]
                    valid = nl >= thrv
                    idx = lb * 16 + iota
                    if j % 2 == 0:
                        plsc.addupdate_scatter(sa_v, [idx], nl, mask=valid)
                    else:
                        plsc.addupdate_scatter(sb_v, [idx], nl, mask=valid)
                return 0
            lax.fori_loop(0, ROWS, row, 0)
            return 0
        lax.fori_loop(0, 128 // ROWS, outer, 0)

        def merge(i, _):
            sa_v[pl.ds(i * 16, 16)] += sb_v[pl.ds(i * 16, 16)]
            return 0
        lax.fori_loop(0, _CPAD, merge, 0, unroll=8)

        pltpu.sync_copy(sa_v, sum_out.at[wid])

    return functools.partial(
        pl.kernel,
        out_type=jax.ShapeDtypeStruct((_NW, _CPAD * 16), jnp.float32),
        mesh=mesh,
        scratch_types=[
            pltpu.VMEM((ROWS, 512), jnp.int32),
            pltpu.VMEM((ROWS, 512), jnp.float32),
            pltpu.VMEM((128,), jnp.float32),
            pltpu.VMEM((_CPAD * 16,), jnp.float32),
            pltpu.VMEM((_CPAD * 16,), jnp.float32),
        ],
        compiler_params=pltpu.CompilerParams(needs_layout_passes=False),
        interpret=interpret,
    )(body)


def _finalize(*, interpret=False):
    def body(cnt_ref, sum_ref, out_ref):
        c = jnp.sum(cnt_ref[...], axis=0)
        s = jnp.sum(sum_ref[...], axis=0)
        ccls = jnp.sum(c, axis=1, keepdims=True)
        scls = jnp.sum(s, axis=1, keepdims=True)
        total = jnp.sum(ccls)
        prop = ccls / total
        wts = 1.0 / jnp.log(1.02 + prop)
        num = jnp.sum(wts * scls)
        den = jnp.sum(wts * ccls)
        out_ref[0, 0] = num / den

    return pl.pallas_call(
        body,
        in_specs=[
            pl.BlockSpec(memory_space=pltpu.MemorySpace.VMEM),
            pl.BlockSpec(memory_space=pltpu.MemorySpace.VMEM),
        ],
        out_specs=pl.BlockSpec(memory_space=pltpu.MemorySpace.SMEM),
        out_shape=jax.ShapeDtypeStruct((1, 1), jnp.float32),
        interpret=interpret,
    )


def _finalize_common(*, interpret=False):
    def body(acc_ref, sum_ref, out_ref):
        ccls = jnp.sum(acc_ref[...], axis=1, keepdims=True)
        s = jnp.sum(sum_ref[...], axis=0)
        scls = jnp.sum(s, axis=1, keepdims=True)
        total = jnp.sum(ccls)
        prop = ccls / total
        wts = 1.0 / jnp.log(1.02 + prop)
        num = jnp.sum(wts * scls)
        den = jnp.sum(wts * ccls)
        out_ref[0, 0] = num / den

    return pl.pallas_call(
        body,
        in_specs=[
            pl.BlockSpec(memory_space=pltpu.MemorySpace.VMEM),
            pl.BlockSpec(memory_space=pltpu.MemorySpace.VMEM),
        ],
        out_specs=pl.BlockSpec(memory_space=pltpu.MemorySpace.SMEM),
        out_shape=jax.ShapeDtypeStruct((1, 1), jnp.float32),
        interpret=interpret,
    )


def _ohem_loss(logits, labels, *, interpret=False, sc_interpret=False):
    picks2d, nll2d, cnt07, acc07 = _phase_a(logits, labels,
                                            interpret=interpret)
    picks = picks2d.reshape(-1)
    labs2d = labels.reshape(8 * 512, 512)
    nll2d_m = nll2d.reshape(8 * 512, 512)

    def _radix_path(picks):
        state0 = jnp.zeros((8, 128), jnp.int32).at[1, :].set(_N_MIN)
        h1 = _hist_pass(21, _NB1, None, interpret=sc_interpret)(picks)
        st1 = _find_bin(11, interpret=interpret)(
            h1.reshape(_NW, 16, 128), state0)
        h2 = _hist_pass(10, _NB1, 21, interpret=sc_interpret)(picks, st1)
        st2 = _find_bin(11, interpret=interpret)(
            h2.reshape(_NW, 16, 128), st1)
        h3 = _hist_pass(0, _NB3, 10, interpret=sc_interpret)(picks, st2)
        return _find_bin(10, final=True, interpret=interpret)(
            h3.reshape(_NW, 8, 128), st2)

    def _common(_):
        thr = jnp.full((8, 128), -jnp.log(jnp.float32(_THRESH)), jnp.float32)
        sums = _class_sums_only(interpret=sc_interpret)(labs2d, nll2d_m, thr)
        return _finalize_common(interpret=interpret)(
            acc07, sums.reshape(_NW, _CPAD, 16))

    def _rare(_):
        thr = _radix_path(picks)
        cnt, sums = _class_sums(interpret=sc_interpret)(labs2d, nll2d_m, thr)
        return _finalize(interpret=interpret)(
            cnt.reshape(_NW, _CPAD, 16), sums.reshape(_NW, _CPAD, 16))

    loss = lax.cond(cnt07[0, 0] >= jnp.float32(_N_MIN + 1), _common, _rare, 0)
    return loss[0, 0]


def kernel(logits, labels):
    return _ohem_loss(logits, labels)


# speedup vs baseline: 1.0927x; 1.0927x over previous
import functools

import jax
import jax.numpy as jnp
from jax import lax
from jax.experimental import pallas as pl
from jax.experimental.pallas import tpu as pltpu
from jax.experimental.pallas import tpu_sc as plsc

_THRESH = 0.7
_N_MIN = 131072
_N_CLASSES = 19
_NPIX = 8 * 512 * 512
_NW = 32
_CHUNK = _NPIX // _NW
_NB1 = 2048
_NB3 = 1024
_CPAD = 32
_TH07_BITS = 0x3F333333


def _phase_a(logits, labels, *, interpret=False):
    N, C, H, W = logits.shape
    BH = 64
    grid = (N, H // BH)

    def body(lg_ref, lb_ref, pick_ref, nll_ref, c07_ref):
        x = lg_ref[0]
        lab = lb_ref[0]
        m = jnp.max(x, axis=0)
        e = jnp.exp(x - m[None])
        s = jnp.sum(e, axis=0)
        cls = lax.broadcasted_iota(jnp.int32, (C, BH, W), 0)
        onehot = cls == lab[None]
        picked = jnp.sum(jnp.where(onehot, x, 0.0), axis=0)
        nll = (m - picked) + jnp.log(s)
        pick = jnp.exp(-nll)
        pick_ref[0] = lax.bitcast_convert_type(pick, jnp.int32)
        nll_ref[0] = nll
        nle = jnp.sum((pick <= jnp.float32(_THRESH)).astype(jnp.float32))

        @pl.when((pl.program_id(0) == 0) & (pl.program_id(1) == 0))
        def _():
            c07_ref[0, 0] = 0.0
        c07_ref[0, 0] += nle

    return pl.pallas_call(
        body,
        grid=grid,
        in_specs=[
            pl.BlockSpec((1, C, BH, W), lambda n, h: (n, 0, h, 0)),
            pl.BlockSpec((1, BH, W), lambda n, h: (n, h, 0)),
        ],
        out_specs=[
            pl.BlockSpec((1, BH, W), lambda n, h: (n, h, 0)),
            pl.BlockSpec((1, BH, W), lambda n, h: (n, h, 0)),
            pl.BlockSpec(memory_space=pltpu.MemorySpace.SMEM,
                         index_map=lambda n, h: (0, 0)),
        ],
        out_shape=[
            jax.ShapeDtypeStruct((N, H, W), jnp.int32),
            jax.ShapeDtypeStruct((N, H, W), jnp.float32),
            jax.ShapeDtypeStruct((1, 1), jnp.float32),
        ],
        interpret=interpret,
    )(logits, labels)


def _hist_pass(shift, nbins, prev_shift, *, interpret=False):
    mesh = plsc.VectorSubcoreMesh(core_axis_name="c", subcore_axis_name="s", num_cores=2, num_subcores=16)
    masked = prev_shift is not None

    scratch = [
        pltpu.VMEM((_CHUNK,), jnp.int32),
        pltpu.VMEM((16 * nbins,), jnp.int32),
        pltpu.VMEM((nbins,), jnp.int32),
    ]
    if masked:
        scratch.append(pltpu.VMEM((128,), jnp.int32))

    def body(picks_hbm, *rest):
        if masked:
            state_hbm, out_hbm, chunk_v, hist_v, row_v, pref_v = rest
        else:
            out_hbm, chunk_v, hist_v, row_v = rest
        wid = lax.axis_index("s") * 2 + lax.axis_index("c")
        base = wid * _CHUNK
        pltpu.sync_copy(picks_hbm.at[pl.ds(base, _CHUNK)], chunk_v)
        if masked:
            pltpu.sync_copy(state_hbm.at[0], pref_v)
            prefv = pref_v[pl.ds(0, 16)]

        zeros16 = jnp.zeros((16,), jnp.int32)

        def zbody(j, _):
            hist_v[pl.ds(j * 16, 16)] = zeros16
            return 0
        lax.fori_loop(0, (16 * nbins) // 16, zbody, 0, unroll=8)

        iota = lax.iota(jnp.int32, 16)
        lane_off = iota * nbins
        ones16 = jnp.ones((16,), jnp.int32)
        shift_v = jnp.full((16,), shift, jnp.int32)
        maskbins = jnp.full((16,), nbins - 1, jnp.int32)
        if masked:
            pshift_v = jnp.full((16,), prev_shift, jnp.int32)

        def hbody(i, _):
            bits = chunk_v[pl.ds(i * 16, 16)]
            b = (lax.shift_right_logical(bits, shift_v) & maskbins) + lane_off
            if masked:
                m = lax.shift_right_logical(bits, pshift_v) == prefv
                plsc.addupdate_scatter(hist_v, [b], ones16, mask=m)
            else:
                plsc.addupdate_scatter(hist_v, [b], ones16)
            return 0
        lax.fori_loop(0, _CHUNK // 16, hbody, 0, unroll=8)

        def sbody(j, _):
            acc = hist_v[pl.ds(j * 16, 16)]
            for l in range(1, 16):
                acc = acc + hist_v[pl.ds(l * nbins + j * 16, 16)]
            row_v[pl.ds(j * 16, 16)] = acc
            return 0
        lax.fori_loop(0, nbins // 16, sbody, 0, unroll=4)

        pltpu.sync_copy(row_v, out_hbm.at[wid])

    return functools.partial(
        pl.kernel,
        out_type=jax.ShapeDtypeStruct((_NW, nbins), jnp.int32),
        mesh=mesh,
        scratch_types=scratch,
        compiler_params=pltpu.CompilerParams(needs_layout_passes=False),
        interpret=interpret,
    )(body)


def _find_bin(width, *, final=False, interpret=False):
    def body(hist_ref, st_ref, out_ref):
        hf = hist_ref[...].astype(jnp.float32)
        h = jnp.sum(hf, axis=0)
        R = h.shape[0]
        r_i = st_ref[1, 0]
        p_i = st_ref[0, 0]
        r_f = r_i.astype(jnp.float32)
        tri = (lax.broadcasted_iota(jnp.int32, (128, 128), 0)
               <= lax.broadcasted_iota(jnp.int32, (128, 128), 1))
        rowcum = jax.lax.dot(h, tri.astype(jnp.float32),
                             preferred_element_type=jnp.float32)
        rowtot = rowcum[:, 127:128]
        stri = (lax.broadcasted_iota(jnp.int32, (R, R), 1)
                < lax.broadcasted_iota(jnp.int32, (R, R), 0))
        prev = jax.lax.dot(stri.astype(jnp.float32), rowtot,
                           preferred_element_type=jnp.float32)
        c = rowcum + prev
        mask = c <= r_f
        bin_i = jnp.sum(mask.astype(jnp.int32))
        cbefore = jnp.max(jnp.where(mask, c, 0.0))
        newrank = r_i - cbefore.astype(jnp.int32)
        newprefix = lax.shift_left(p_i, width) | bin_i
        if final:
            q = lax.bitcast_convert_type(newprefix, jnp.float32)
            th = jnp.maximum(q, jnp.float32(_THRESH))
            thr = -jnp.log(th)
            for j in range(16):
                out_ref[0, j] = thr
        else:
            for j in range(16):
                out_ref[0, j] = newprefix
                out_ref[1, j] = newrank

    out_dtype = jnp.float32 if final else jnp.int32
    return pl.pallas_call(
        body,
        in_specs=[
            pl.BlockSpec(memory_space=pltpu.MemorySpace.VMEM),
            pl.BlockSpec(memory_space=pltpu.MemorySpace.SMEM),
        ],
        out_specs=pl.BlockSpec(memory_space=pltpu.MemorySpace.SMEM),
        out_shape=jax.ShapeDtypeStruct((8, 128), out_dtype),
        interpret=interpret,
    )


def _class_sums(*, interpret=False):
    mesh = plsc.VectorSubcoreMesh(core_axis_name="c", subcore_axis_name="s",
                                  num_cores=2, num_subcores=16)
    ROWS = 32

    def body(labels_hbm, nll_hbm, thr_hbm,
             cnt_out, sum_out,
             lab_v, nll_v, thr_v, ca_v, cb_v, sa_v, sb_v):
        wid = lax.axis_index("s") * 2 + lax.axis_index("c")
        rbase = wid * 128
        pltpu.sync_copy(thr_hbm.at[0], thr_v)
        thrv = thr_v[pl.ds(0, 16)]

        zeros16 = jnp.zeros((16,), jnp.float32)
        for i in range(_CPAD):
            ca_v[pl.ds(i * 16, 16)] = zeros16
            cb_v[pl.ds(i * 16, 16)] = zeros16
            sa_v[pl.ds(i * 16, 16)] = zeros16
            sb_v[pl.ds(i * 16, 16)] = zeros16

        iota = lax.iota(jnp.int32, 16)
        ones16 = jnp.ones((16,), jnp.float32)

        def outer(k, _):
            pltpu.sync_copy(labels_hbm.at[pl.ds(rbase + k * ROWS, ROWS)], lab_v)
            pltpu.sync_copy(nll_hbm.at[pl.ds(rbase + k * ROWS, ROWS)], nll_v)

            def row(r, _):
                for j in range(32):
                    lb = lab_v[r, pl.ds(j * 16, 16)]
                    nl = nll_v[r, pl.ds(j * 16, 16)]
                    valid = nl >= thrv
                    idx = lb * 16 + iota
                    if j % 2 == 0:
                        plsc.addupdate_scatter(ca_v, [idx], ones16,
                                               mask=valid)
                        plsc.addupdate_scatter(sa_v, [idx], nl, mask=valid)
                    else:
                        plsc.addupdate_scatter(cb_v, [idx], ones16,
                                               mask=valid)
                        plsc.addupdate_scatter(sb_v, [idx], nl, mask=valid)
                return 0
            lax.fori_loop(0, ROWS, row, 0)
            return 0
        lax.fori_loop(0, 128 // ROWS, outer, 0)

        def merge(i, _):
            ca_v[pl.ds(i * 16, 16)] += cb_v[pl.ds(i * 16, 16)]
            sa_v[pl.ds(i * 16, 16)] += sb_v[pl.ds(i * 16, 16)]
            return 0
        lax.fori_loop(0, _CPAD, merge, 0, unroll=8)

        pltpu.sync_copy(ca_v, cnt_out.at[wid])
        pltpu.sync_copy(sa_v, sum_out.at[wid])

    return functools.partial(
        pl.kernel,
        out_type=[
            jax.ShapeDtypeStruct((_NW, _CPAD * 16), jnp.float32),
            jax.ShapeDtypeStruct((_NW, _CPAD * 16), jnp.float32),
        ],
        mesh=mesh,
        scratch_types=[
            pltpu.VMEM((ROWS, 512), jnp.int32),
            pltpu.VMEM((ROWS, 512), jnp.float32),
            pltpu.VMEM((128,), jnp.float32),
            pltpu.VMEM((_CPAD * 16,), jnp.float32),
            pltpu.VMEM((_CPAD * 16,), jnp.float32),
            pltpu.VMEM((_CPAD * 16,), jnp.float32),
            pltpu.VMEM((_CPAD * 16,), jnp.float32),
        ],
        compiler_params=pltpu.CompilerParams(needs_layout_passes=False),
        interpret=interpret,
    )(body)


def _finalize(*, interpret=False):
    def body(cnt_ref, sum_ref, out_ref):
        c = jnp.sum(cnt_ref[...], axis=0)
        s = jnp.sum(sum_ref[...], axis=0)
        ccls = jnp.sum(c, axis=1, keepdims=True)
        scls = jnp.sum(s, axis=1, keepdims=True)
        total = jnp.sum(ccls)
        prop = ccls / total
        wts = 1.0 / jnp.log(1.02 + prop)
        num = jnp.sum(wts * scls)
        den = jnp.sum(wts * ccls)
        out_ref[0, 0] = num / den

    return pl.pallas_call(
        body,
        in_specs=[
            pl.BlockSpec(memory_space=pltpu.MemorySpace.VMEM),
            pl.BlockSpec(memory_space=pltpu.MemorySpace.VMEM),
        ],
        out_specs=pl.BlockSpec(memory_space=pltpu.MemorySpace.SMEM),
        out_shape=jax.ShapeDtypeStruct((1, 1), jnp.float32),
        interpret=interpret,
    )


def _ohem_loss(logits, labels, *, interpret=False, sc_interpret=False):
    picks2d, nll2d, cnt07 = _phase_a(logits, labels, interpret=interpret)
    picks = picks2d.reshape(-1)
    labs2d = labels.reshape(8 * 512, 512)
    nll2d_m = nll2d.reshape(8 * 512, 512)

    def _radix_path(picks):
        state0 = jnp.zeros((8, 128), jnp.int32).at[1, :].set(_N_MIN)
        h1 = _hist_pass(21, _NB1, None, interpret=sc_interpret)(picks)
        st1 = _find_bin(11, interpret=interpret)(
            h1.reshape(_NW, 16, 128), state0)
        h2 = _hist_pass(10, _NB1, 21, interpret=sc_interpret)(picks, st1)
        st2 = _find_bin(11, interpret=interpret)(
            h2.reshape(_NW, 16, 128), st1)
        h3 = _hist_pass(0, _NB3, 10, interpret=sc_interpret)(picks, st2)
        return _find_bin(10, final=True, interpret=interpret)(
            h3.reshape(_NW, 8, 128), st2)

    thr = lax.cond(
        cnt07[0, 0] >= jnp.float32(_N_MIN + 1),
        lambda p: jnp.full((8, 128), -jnp.log(jnp.float32(_THRESH)),
                           jnp.float32),
        _radix_path,
        picks)

    cnt, sums = _class_sums(interpret=sc_interpret)(labs2d, nll2d_m, thr)
    loss = _finalize(interpret=interpret)(
        cnt.reshape(_NW, _CPAD, 16), sums.reshape(_NW, _CPAD, 16))
    return loss[0, 0]


def kernel(logits, labels):
    return _ohem_loss(logits, labels)
